# Initial kernel scaffold; baseline (speedup 1.0000x reference)
import jax
import jax.numpy as jnp
from jax.experimental import pallas as pl


def kernel(inputs, tables):
    raise NotImplementedError("write your pallas kernel here")


# speedup vs baseline: 1.3299x; 1.3299x over previous
import functools

import jax
import jax.numpy as jnp
from jax import lax
from jax.experimental import pallas as pl
from jax.experimental.pallas import tpu as pltpu
from jax.experimental.pallas import tpu_sc as plsc

_NUM_LEVELS = 16
_LOG2_HASH = 19
_MASK = (1 << _LOG2_HASH) - 1
_B = 524288
_C0, _C1, _C2 = 73856093, 19349663, 83492791

_NC, _NS, _L = 2, 16, 16
_NW = _NC * _NS
_PER_W = _B // _NW
_CHUNK = 512
_NCHUNK = _PER_W // _CHUNK
_NV = _CHUNK // _L
_NG = _CHUNK * 2 // _L
_OUTW = 2 * _NUM_LEVELS


def _table_size(level):
    resolution = 16 * 2 ** level
    return min(1 << _LOG2_HASH, (resolution + 1) ** 3)


def _hash_kernel(inp_hbm, *rest):
    tables = rest[:_NUM_LEVELS]
    out_hbm = rest[_NUM_LEVELS]
    (inp_v, idx_v, sub_v, rows_v, out_v,
     sem_in, sem_g, sem_out) = rest[_NUM_LEVELS + 1:]

    wid = lax.axis_index("c") * jnp.int32(_NS) + lax.axis_index("s")
    iota = lax.iota(jnp.int32, _L)
    half = iota >> jnp.int32(1)
    bit = iota & jnp.int32(1)
    bp = half * jnp.int32(_OUTW) + bit
    lconst = [jnp.full((_L,), l, jnp.int32) for l in range(_NUM_LEVELS)]

    def run_chunk(_, base):
        base = pl.multiple_of(base, _CHUNK)
        pltpu.async_copy(inp_hbm.at[pl.ds(base, _CHUNK)], inp_v, sem_in).wait()

        def compute(_, off):
            pvec = off + iota
            ks = []
            for d in range(3):
                col = plsc.load_gather(inp_v, [pvec, lconst[d]])
                s = (col + jnp.float32(1.0)) * jnp.float32(262144.0)
                k = s.astype(jnp.int32)
                ks.append(jnp.clip(k, jnp.int32(0), jnp.int32(_MASK)))
            kx, ky, kz = ks
            for l in range(_NUM_LEVELS):
                sh = jnp.int32(_NUM_LEVELS - 1 - l)
                h = ((kx >> sh) * jnp.int32(_C0)
                     + (ky >> sh) * jnp.int32(_C1)
                     + (kz >> sh) * jnp.int32(_C2)) & jnp.int32(_MASK)
                sz = _table_size(l)
                if sz <= _MASK:
                    h = jnp.minimum(h, jnp.int32(sz - 1))
                idx_v[l, pl.ds(off, _L)] = h >> jnp.int32(2)
                sub_v[l, pl.ds(off, _L)] = (h & jnp.int32(3)) * jnp.int32(2)
            return off + jnp.int32(_L)

        lax.fori_loop(0, _NV, compute, jnp.int32(0))

        copies = [
            pltpu.async_copy(tables[l].at[idx_v.at[jnp.int32(l)]],
                             rows_v.at[jnp.int32(l)], sem_g)
            for l in range(_NUM_LEVELS)
        ]
        for cp in copies:
            cp.wait()

        def interleave(_, carry):
            j8, oidx = carry
            pvec = j8 + half
            for l in range(_NUM_LEVELS):
                sub2 = plsc.load_gather(sub_v, [lconst[l], pvec])
                val = plsc.load_gather(rows_v, [lconst[l], pvec, sub2 + bit])
                plsc.store_scatter(out_v, [oidx + jnp.int32(2 * l)], val)
            return j8 + jnp.int32(8), oidx + jnp.int32(8 * _OUTW)

        lax.fori_loop(0, _NG, interleave, (jnp.int32(0), bp))

        pltpu.async_copy(
            out_v, out_hbm.at[pl.ds(base * jnp.int32(_OUTW), _CHUNK * _OUTW)],
            sem_out).wait()
        return base + jnp.int32(_CHUNK)

    lax.fori_loop(0, _NCHUNK, run_chunk, wid * jnp.int32(_PER_W))


def kernel(inputs, tables):
    tabs = []
    for t in tables:
        s = t.shape[0]
        if s % 4:
            t = jnp.pad(t, ((0, 4 - s % 4), (0, 0)))
        tabs.append(t.reshape(-1, 8))
    mesh = plsc.VectorSubcoreMesh(core_axis_name="c", subcore_axis_name="s")
    k = functools.partial(
        pl.kernel,
        mesh=mesh,
        compiler_params=pltpu.CompilerParams(needs_layout_passes=False,
                                             use_tc_tiling_on_sc=False),
        out_type=jax.ShapeDtypeStruct((_B * _OUTW,), jnp.float32),
        scratch_types=[
            pltpu.VMEM((_CHUNK, 3), jnp.float32),
            pltpu.VMEM((_NUM_LEVELS, _CHUNK), jnp.int32),
            pltpu.VMEM((_NUM_LEVELS, _CHUNK), jnp.int32),
            pltpu.VMEM((_NUM_LEVELS, _CHUNK, 8), jnp.float32),
            pltpu.VMEM((_CHUNK * _OUTW,), jnp.float32),
            pltpu.SemaphoreType.DMA,
            pltpu.SemaphoreType.DMA,
            pltpu.SemaphoreType.DMA,
        ],
    )(_hash_kernel)
    out = k(inputs, *tabs)
    return out.reshape(_B, _OUTW)


# speedup vs baseline: 1.3470x; 1.0129x over previous
import functools

import jax
import jax.numpy as jnp
from jax import lax
from jax.experimental import pallas as pl
from jax.experimental.pallas import tpu as pltpu
from jax.experimental.pallas import tpu_sc as plsc

_NUM_LEVELS = 16
_LOG2_HASH = 19
_MASK = (1 << _LOG2_HASH) - 1
_B = 524288
_C0, _C1, _C2 = 73856093, 19349663, 83492791

_NC, _NS, _L = 2, 16, 16
_NW = _NC * _NS
_PER_W = _B // _NW
_CHUNK = 256
_NCHUNK = _PER_W // _CHUNK
_NPAIR = _NCHUNK // 2
_NV = _CHUNK // _L
_NG = _CHUNK * 2 // _L
_OUTW = 2 * _NUM_LEVELS


def _table_size(level):
    resolution = 16 * 2 ** level
    return min(1 << _LOG2_HASH, (resolution + 1) ** 3)


def _hash_kernel(inp_hbm, *rest):
    tables = rest[:_NUM_LEVELS]
    out_hbm = rest[_NUM_LEVELS]
    (inp0, inp1, idx0, idx1, sub0, sub1, rows0, rows1, out0, out1,
     sem_in, sem_g0, sem_g1, sem_o0, sem_o1) = rest[_NUM_LEVELS + 1:]

    wid = lax.axis_index("c") * jnp.int32(_NS) + lax.axis_index("s")
    wbase = wid * jnp.int32(_PER_W)
    iota = lax.iota(jnp.int32, _L)
    half = iota >> jnp.int32(1)
    bit = iota & jnp.int32(1)
    bp = half * jnp.int32(_OUTW) + bit
    lconst = [jnp.full((_L,), l, jnp.int32) for l in range(_NUM_LEVELS)]

    def in_copy(base, buf):
        base = pl.multiple_of(base, _CHUNK)
        return pltpu.make_async_copy(inp_hbm.at[pl.ds(base, _CHUNK)], buf,
                                     sem_in)

    def out_copy(base, buf, sem):
        base = pl.multiple_of(base, _CHUNK)
        return pltpu.make_async_copy(
            buf, out_hbm.at[pl.ds(base * jnp.int32(_OUTW), _CHUNK * _OUTW)],
            sem)

    def g_copies(idx_v, rows_v, sem):
        return [pltpu.make_async_copy(tables[l].at[idx_v.at[jnp.int32(l)]],
                                      rows_v.at[jnp.int32(l)], sem)
                for l in range(_NUM_LEVELS)]

    def compute_idx(inp_v, idx_v, sub_v):
        def body(_, off):
            pvec = off + iota
            ks = []
            for d in range(3):
                col = plsc.load_gather(inp_v, [pvec, lconst[d]])
                s = (col + jnp.float32(1.0)) * jnp.float32(262144.0)
                k = s.astype(jnp.int32)
                ks.append(jnp.clip(k, jnp.int32(0), jnp.int32(_MASK)))
            kx, ky, kz = ks
            for l in range(_NUM_LEVELS):
                sh = jnp.int32(_NUM_LEVELS - 1 - l)
                h = ((kx >> sh) * jnp.int32(_C0)
                     + (ky >> sh) * jnp.int32(_C1)
                     + (kz >> sh) * jnp.int32(_C2)) & jnp.int32(_MASK)
                sz = _table_size(l)
                if sz <= _MASK:
                    h = jnp.minimum(h, jnp.int32(sz - 1))
                idx_v[l, pl.ds(off, _L)] = h >> jnp.int32(2)
                sub_v[l, pl.ds(off, _L)] = (h & jnp.int32(3)) * jnp.int32(2)
            return off + jnp.int32(_L)

        lax.fori_loop(0, _NV, body, jnp.int32(0))

    def interleave(sub_v, rows_v, out_v):
        def body(_, carry):
            j8, oidx = carry
            pvec = j8 + half
            for l in range(_NUM_LEVELS):
                sub2 = plsc.load_gather(sub_v, [lconst[l], pvec])
                val = plsc.load_gather(rows_v, [lconst[l], pvec, sub2 + bit])
                plsc.store_scatter(out_v, [oidx + jnp.int32(2 * l)], val)
            return j8 + jnp.int32(8), oidx + jnp.int32(8 * _OUTW)

        lax.fori_loop(0, _NG, body, (jnp.int32(0), bp))

    in_copy(wbase, inp0).start()

    def pair(i, base):
        base = pl.multiple_of(base, _CHUNK)
        b1 = base + jnp.int32(_CHUNK)

        in_copy(base, inp0).wait()
        compute_idx(inp0, idx0, sub0)
        in_copy(b1, inp1).start()
        for cp in g_copies(idx0, rows0, sem_g0):
            cp.start()

        @pl.when(i >= jnp.int32(2))
        def _():
            out_copy(base, out1, sem_o1).wait()

        @pl.when(i >= jnp.int32(1))
        def _():
            for cp in g_copies(idx1, rows1, sem_g1):
                cp.wait()
            interleave(sub1, rows1, out1)
            out_copy(base - jnp.int32(_CHUNK), out1, sem_o1).start()

        in_copy(b1, inp1).wait()
        compute_idx(inp1, idx1, sub1)

        @pl.when(i < jnp.int32(_NPAIR - 1))
        def _():
            in_copy(b1 + jnp.int32(_CHUNK), inp0).start()

        for cp in g_copies(idx1, rows1, sem_g1):
            cp.start()

        @pl.when(i >= jnp.int32(1))
        def _():
            out_copy(base, out0, sem_o0).wait()

        for cp in g_copies(idx0, rows0, sem_g0):
            cp.wait()
        interleave(sub0, rows0, out0)
        out_copy(base, out0, sem_o0).start()
        return base + jnp.int32(2 * _CHUNK)

    def pair_loop(i, carry):
        cnt, base = carry
        return cnt + jnp.int32(1), pair(cnt, base)

    lax.fori_loop(0, _NPAIR, pair_loop, (jnp.int32(0), wbase))

    last = wbase + jnp.int32((_NCHUNK - 1) * _CHUNK)
    out_copy(last, out1, sem_o1).wait()
    for cp in g_copies(idx1, rows1, sem_g1):
        cp.wait()
    interleave(sub1, rows1, out1)
    out_copy(last, out1, sem_o1).start()
    out_copy(last, out0, sem_o0).wait()
    out_copy(last, out1, sem_o1).wait()


def kernel(inputs, tables):
    tabs = []
    for t in tables:
        s = t.shape[0]
        if s % 4:
            t = jnp.pad(t, ((0, 4 - s % 4), (0, 0)))
        tabs.append(t.reshape(-1, 8))
    mesh = plsc.VectorSubcoreMesh(core_axis_name="c", subcore_axis_name="s")
    k = functools.partial(
        pl.kernel,
        mesh=mesh,
        compiler_params=pltpu.CompilerParams(needs_layout_passes=False,
                                             use_tc_tiling_on_sc=False),
        out_type=jax.ShapeDtypeStruct((_B * _OUTW,), jnp.float32),
        scratch_types=[
            pltpu.VMEM((_CHUNK, 3), jnp.float32),
            pltpu.VMEM((_CHUNK, 3), jnp.float32),
            pltpu.VMEM((_NUM_LEVELS, _CHUNK), jnp.int32),
            pltpu.VMEM((_NUM_LEVELS, _CHUNK), jnp.int32),
            pltpu.VMEM((_NUM_LEVELS, _CHUNK), jnp.int32),
            pltpu.VMEM((_NUM_LEVELS, _CHUNK), jnp.int32),
            pltpu.VMEM((_NUM_LEVELS, _CHUNK, 8), jnp.float32),
            pltpu.VMEM((_NUM_LEVELS, _CHUNK, 8), jnp.float32),
            pltpu.VMEM((_CHUNK * _OUTW,), jnp.float32),
            pltpu.VMEM((_CHUNK * _OUTW,), jnp.float32),
            pltpu.SemaphoreType.DMA,
            pltpu.SemaphoreType.DMA,
            pltpu.SemaphoreType.DMA,
            pltpu.SemaphoreType.DMA,
            pltpu.SemaphoreType.DMA,
        ],
    )(_hash_kernel)
    out = k(inputs, *tabs)
    return out.reshape(_B, _OUTW)


# speedup vs baseline: 3.5472x; 2.6334x over previous
import functools

import jax
import jax.numpy as jnp
from jax import lax
from jax.experimental import pallas as pl
from jax.experimental.pallas import tpu as pltpu
from jax.experimental.pallas import tpu_sc as plsc

_NUM_LEVELS = 16
_LOG2_HASH = 19
_MASK = (1 << _LOG2_HASH) - 1
_B = 524288
_C0, _C1, _C2 = 73856093, 19349663, 83492791

_NC, _NS, _L = 2, 16, 16
_NW = _NC * _NS
_PER_W = _B // _NW
_CHUNK = 128
_NCHUNK = _PER_W // _CHUNK
_NPAIR = _NCHUNK // 2
_NV = _CHUNK // _L
_NG = _CHUNK * 2 // _L
_OUTW = 2 * _NUM_LEVELS


def _table_size(level):
    resolution = 16 * 2 ** level
    return min(1 << _LOG2_HASH, (resolution + 1) ** 3)


def _hash_kernel(inp_hbm, *rest):
    tables = rest[:_NUM_LEVELS]
    out_hbm = rest[_NUM_LEVELS]
    (inp0, idx0, idx1, sub0, sub1, rows0, rows1, out0, out1,
     sem_in, sem_g0, sem_g1, sem_o0, sem_o1) = rest[_NUM_LEVELS + 1:]

    wid = lax.axis_index("c") * jnp.int32(_NS) + lax.axis_index("s")
    wbase = wid * jnp.int32(_PER_W)
    iota = lax.iota(jnp.int32, _L)
    half = iota >> jnp.int32(1)
    bit = iota & jnp.int32(1)
    lconst = [jnp.full((_L,), l, jnp.int32) for l in range(_NUM_LEVELS)]

    def in_copy(base, buf):
        base = pl.multiple_of(base, _CHUNK)
        return pltpu.make_async_copy(
            inp_hbm.at[pl.ds(base >> jnp.int32(5), 8)], buf, sem_in)

    def out_copies(base, buf, sem):
        base = pl.multiple_of(base, _CHUNK)
        p_off = base * jnp.int32(8)
        return [pltpu.make_async_copy(
                    buf.at[pl.ds(r * 1024, 1024)],
                    out_hbm.at[pl.ds(p_off + jnp.int32(r * _B * 8), 1024)],
                    sem)
                for r in range(4)]

    def g_copies(idx_v, rows_v, sem):
        return [pltpu.make_async_copy(
                    tables[l].at[idx_v.at[jnp.int32(l), jnp.int32(c)]],
                    rows_v.at[jnp.int32(l), jnp.int32(c)], sem)
                for l in range(_NUM_LEVELS) for c in range(2)]

    def compute_idx(inp_v, row0, idx_v, sub_v):
        def body(_, off):
            ks = []
            for d in range(3):
                col = inp_v[row0 + d, pl.ds(off, _L)]
                s = (col + jnp.float32(1.0)) * jnp.float32(262144.0)
                k = s.astype(jnp.int32)
                ks.append(jnp.clip(k, jnp.int32(0), jnp.int32(_MASK)))
            kx, ky, kz = ks
            for l in range(_NUM_LEVELS):
                sh = jnp.int32(_NUM_LEVELS - 1 - l)
                h = ((kx >> sh) * jnp.int32(_C0)
                     + (ky >> sh) * jnp.int32(_C1)
                     + (kz >> sh) * jnp.int32(_C2)) & jnp.int32(_MASK)
                sz = _table_size(l)
                if sz <= _MASK:
                    h = jnp.minimum(h, jnp.int32(sz - 1))
                g0 = ((h >> jnp.int32(7)) << jnp.int32(5)) + (
                    (h & jnp.int32(127)) >> jnp.int32(3))
                idx_v[l, 0, pl.ds(off, _L)] = g0
                idx_v[l, 1, pl.ds(off, _L)] = g0 + jnp.int32(16)
                sub_v[l, pl.ds(off, _L)] = h & jnp.int32(7)
            return off + jnp.int32(_L)

        lax.fori_loop(0, _NV, body, jnp.int32(0))

    def interleave(sub_v, rows_v, out_v):
        b128 = bit * jnp.int32(128)

        def body(_, j8):
            pvec = j8 + half
            for l in range(_NUM_LEVELS):
                kl = (l >> 2) * 1024 + ((2 * l) & 7) * 128
                sub = plsc.load_gather(sub_v, [lconst[l], pvec])
                val = plsc.load_gather(rows_v, [lconst[l], bit, pvec, sub])
                plsc.store_scatter(out_v, [pvec + b128 + jnp.int32(kl)], val)
            return j8 + jnp.int32(8)

        lax.fori_loop(0, _NG, body, jnp.int32(0))

    def pair(i, base):
        base = pl.multiple_of(base, _CHUNK)

        cp = in_copy(base, inp0)
        cp.start()
        cp.wait()
        compute_idx(inp0, 0, idx0, sub0)
        for cp in g_copies(idx0, rows0, sem_g0):
            cp.start()

        @pl.when(i >= jnp.int32(2))
        def _():
            for cp in out_copies(base, out1, sem_o1):
                cp.wait()

        @pl.when(i >= jnp.int32(1))
        def _():
            for cp in g_copies(idx1, rows1, sem_g1):
                cp.wait()
            interleave(sub1, rows1, out1)
            for cp in out_copies(base - jnp.int32(_CHUNK), out1, sem_o1):
                cp.start()

        compute_idx(inp0, 4, idx1, sub1)

        for cp in g_copies(idx1, rows1, sem_g1):
            cp.start()

        @pl.when(i >= jnp.int32(1))
        def _():
            for cp in out_copies(base, out0, sem_o0):
                cp.wait()

        for cp in g_copies(idx0, rows0, sem_g0):
            cp.wait()
        interleave(sub0, rows0, out0)
        for cp in out_copies(base, out0, sem_o0):
            cp.start()
        return base + jnp.int32(2 * _CHUNK)

    def pair_loop(i, carry):
        cnt, base = carry
        return cnt + jnp.int32(1), pair(cnt, base)

    lax.fori_loop(0, _NPAIR, pair_loop, (jnp.int32(0), wbase))

    last = wbase + jnp.int32((_NCHUNK - 1) * _CHUNK)
    for cp in out_copies(last, out1, sem_o1):
        cp.wait()
    for cp in g_copies(idx1, rows1, sem_g1):
        cp.wait()
    interleave(sub1, rows1, out1)
    for cp in out_copies(last, out1, sem_o1):
        cp.start()
    for cp in out_copies(last, out0, sem_o0):
        cp.wait()
    for cp in out_copies(last, out1, sem_o1):
        cp.wait()


def kernel(inputs, tables):
    inp_view = (jnp.pad(inputs, ((0, 0), (0, 1)))
                .reshape(_B // 128, 128, 4).transpose(0, 2, 1)
                .reshape(_B // 32, 128))
    tabs = []
    for t in tables:
        s = t.shape[0]
        if s % 128:
            t = jnp.pad(t, ((0, 128 - s % 128), (0, 0)))
        sb = t.shape[0] // 128
        tabs.append(t.reshape(sb, 128, 2).transpose(0, 2, 1).reshape(sb * 32, 8))
    mesh = plsc.VectorSubcoreMesh(core_axis_name="c", subcore_axis_name="s")
    k = functools.partial(
        pl.kernel,
        mesh=mesh,
        compiler_params=pltpu.CompilerParams(needs_layout_passes=False,
                                             use_tc_tiling_on_sc=False),
        out_type=jax.ShapeDtypeStruct((_B * _OUTW,), jnp.float32),
        scratch_types=[
            pltpu.VMEM((8, 128), jnp.float32),
            pltpu.VMEM((_NUM_LEVELS, 2, _CHUNK), jnp.int32),
            pltpu.VMEM((_NUM_LEVELS, 2, _CHUNK), jnp.int32),
            pltpu.VMEM((_NUM_LEVELS, _CHUNK), jnp.int32),
            pltpu.VMEM((_NUM_LEVELS, _CHUNK), jnp.int32),
            pltpu.VMEM((_NUM_LEVELS, 2, _CHUNK, 8), jnp.float32),
            pltpu.VMEM((_NUM_LEVELS, 2, _CHUNK, 8), jnp.float32),
            pltpu.VMEM((_CHUNK * _OUTW,), jnp.float32),
            pltpu.VMEM((_CHUNK * _OUTW,), jnp.float32),
            pltpu.SemaphoreType.DMA,
            pltpu.SemaphoreType.DMA,
            pltpu.SemaphoreType.DMA,
            pltpu.SemaphoreType.DMA,
            pltpu.SemaphoreType.DMA,
        ],
    )(_hash_kernel)
    out = k(inp_view, *tabs)
    return (out.reshape(4, _B // 128, 8, 128)
            .transpose(1, 3, 0, 2).reshape(_B, _OUTW))
